# baseline (speedup 1.0000x reference)
import jax
import jax.numpy as jnp
from jax.experimental import pallas as pl

_EMBED = 256
_S = 8
_ROWS_PER_BLOCK = 1024


def _embed_body(xv, xd, xp, W, bv, bd, bp, out):
    x = jnp.concatenate(
        [xv[...].astype(jnp.bfloat16),
         xd[...].astype(jnp.bfloat16),
         xp[...].astype(jnp.bfloat16)], axis=1)
    dn = (((1,), (1,)), ((), ()))
    acc = jax.lax.dot_general(x, W[...], dn,
                              preferred_element_type=jnp.float32)
    bias = bv[...] + bd[...] + jnp.sum(bp[...], axis=0, keepdims=True)
    out[...] = acc + bias


@jax.jit
def kernel(value, depth, pos, Wv, bv, Wd, bd, Wp, bp):
    B, L = value.shape
    T = L // _S
    N = B * T

    xv = value.reshape(N, _S)
    xd = depth.reshape(N, _S)
    xp = pos.reshape(N, _S * 3)

    Wpp = Wp.transpose(1, 2, 0).reshape(_EMBED, _S * 3)
    W = jnp.concatenate([Wv, Wd, Wpp], axis=1).astype(jnp.bfloat16)

    bv2 = bv.reshape(1, _EMBED)
    bd2 = bd.reshape(1, _EMBED)

    R = _ROWS_PER_BLOCK
    grid = (N // R,)

    out = pl.pallas_call(
        _embed_body,
        grid=grid,
        in_specs=[
            pl.BlockSpec((R, _S), lambda i: (i, 0)),
            pl.BlockSpec((R, _S), lambda i: (i, 0)),
            pl.BlockSpec((R, _S * 3), lambda i: (i, 0)),
            pl.BlockSpec((_EMBED, _S * 5), lambda i: (0, 0)),
            pl.BlockSpec((1, _EMBED), lambda i: (0, 0)),
            pl.BlockSpec((1, _EMBED), lambda i: (0, 0)),
            pl.BlockSpec((3, _EMBED), lambda i: (0, 0)),
        ],
        out_specs=pl.BlockSpec((R, _EMBED), lambda i: (i, 0)),
        out_shape=jax.ShapeDtypeStruct((N, _EMBED), jnp.float32),
    )(xv, xd, xp, W, bv2, bd2, bp)

    return out.reshape(B, T, _EMBED)
